# baseline (speedup 1.0000x reference)
import functools

import jax
import jax.numpy as jnp
from jax import lax
from jax.experimental import pallas as pl
from jax.experimental.pallas import tpu as pltpu
from jax.experimental.pallas import tpu_sc as plsc

B = 1024
T = 200
E = 64
N = B * T

NC = 2
NS = 16
L = 16
NW = NC * NS

PER_W = N // NW
CH = 320
NCHUNK = PER_W // CH
WR = 40


def _sc_embed(xf, tbl3, pos_table):
    mesh = plsc.VectorSubcoreMesh(
        core_axis_name="c", subcore_axis_name="s", num_cores=NC, num_subcores=NS
    )

    @functools.partial(
        pl.kernel,
        out_type=jax.ShapeDtypeStruct((B, T, E), jnp.float32),
        mesh=mesh,
        compiler_params=pltpu.CompilerParams(use_tc_tiling_on_sc=False),
        scratch_types=[
            pltpu.VMEM((CH,), jnp.int32),
            pltpu.VMEM((CH,), jnp.int32),
            pltpu.VMEM((CH,), jnp.int32),
            pltpu.VMEM((CH,), jnp.int32),
            pltpu.VMEM((CH, 2, E), jnp.float32),
            pltpu.VMEM((CH, 2, E), jnp.float32),
            pltpu.VMEM((CH, E), jnp.float32),
            pltpu.VMEM((T, E), jnp.float32),
            pltpu.SemaphoreType.DMA,
            pltpu.SemaphoreType.DMA,
        ],
    )
    def k(x_hbm, tbl_hbm, pos_hbm, out_hbm,
          idx_v0, idx_v1, pidx_v0, pidx_v1, rows_v0, rows_v1,
          out_v, pos_v, gsem0, gsem1):
        idx_b = (idx_v0, idx_v1)
        pidx_b = (pidx_v0, pidx_v1)
        rows_b = (rows_v0, rows_v1)
        gsem_b = (gsem0, gsem1)
        wid = lax.axis_index("c") * NS + lax.axis_index("s")
        base = wid * PER_W
        bbase = wid * (PER_W // T)
        pltpu.sync_copy(pos_hbm, pos_v)

        def stage(i, b):
            off = pl.multiple_of(base + i * CH, 8)
            idx_v = idx_b[b]
            pidx_v = pidx_b[b]
            pltpu.sync_copy(x_hbm.at[pl.ds(off, CH)], idx_v)

            def shift_body(v, _):
                pidx_v[pl.ds(v * L, L)] = lax.shift_right_logical(
                    idx_v[pl.ds(v * L, L)], 1
                )
                return 0

            lax.fori_loop(0, CH // L, shift_body, 0)
            pltpu.async_copy(tbl_hbm.at[pidx_v], rows_b[b], gsem_b[b])

        def consume(i, b):
            idx_v = idx_b[b]
            rows_v = rows_b[b]
            pltpu.make_async_copy(
                tbl_hbm.at[pidx_b[b]], rows_v, gsem_b[b]
            ).wait()
            t0 = lax.rem(i * CH, T)

            def grp_body(m, _):
                iv = idx_v[pl.ds(m * L, L)]
                for r in range(L):
                    tok = m * L + r
                    a = iv[r] & 1
                    tt = t0 + tok
                    tt = jnp.where(tt >= T, tt - T, tt)
                    tt = jnp.where(tt >= T, tt - T, tt)
                    for c in range(E // L):
                        out_v[tok, pl.ds(c * L, L)] = (
                            rows_v[tok, a, pl.ds(c * L, L)]
                            + pos_v[tt, pl.ds(c * L, L)]
                        )
                return 0

            lax.fori_loop(0, CH // L, grp_body, 0)
            for w in range(CH // WR):
                tok0 = i * CH + w * WR
                bq = bbase + tok0 // T
                tq = pl.multiple_of(lax.rem(tok0, T), 8)
                pltpu.sync_copy(
                    out_v.at[pl.ds(w * WR, WR)],
                    out_hbm.at[bq, pl.ds(tq, WR)],
                )

        stage(0, 0)

        def pair_body(g, _):
            i0 = 2 * g
            stage(i0 + 1, 1)
            consume(i0, 0)
            stage(i0 + 2, 0)
            consume(i0 + 1, 1)
            return 0

        lax.fori_loop(0, NCHUNK // 2 - 1, pair_body, 0)
        stage(NCHUNK - 1, 1)
        consume(NCHUNK - 2, 0)
        consume(NCHUNK - 1, 1)

    return k(xf, tbl3, pos_table)


def kernel(x, token_table, pos_table):
    xf = x.reshape(N).astype(jnp.int32)
    tbl3 = token_table.reshape(500000, 2, E)
    return _sc_embed(xf, tbl3, pos_table)


# speedup vs baseline: 2.0721x; 2.0721x over previous
import functools

import jax
import jax.numpy as jnp
from jax import lax
from jax.experimental import pallas as pl
from jax.experimental.pallas import tpu as pltpu
from jax.experimental.pallas import tpu_sc as plsc

B = 1024
T = 200
E = 64
N = B * T

NC = 2
NS = 16
L = 16
NW = NC * NS

PER_W = N // NW
SEQ_PER_CHUNK = 4
CH = SEQ_PER_CHUNK * T
NCHUNK = PER_W // CH


def _sc_embed(xf, token_table, pos_table):
    mesh = plsc.VectorSubcoreMesh(
        core_axis_name="c", subcore_axis_name="s", num_cores=NC, num_subcores=NS
    )

    @functools.partial(
        pl.kernel,
        out_type=jax.ShapeDtypeStruct((B, T, E), jnp.float32),
        mesh=mesh,
        compiler_params=pltpu.CompilerParams(use_tc_tiling_on_sc=False),
        scratch_types=[
            pltpu.VMEM((CH,), jnp.int32),
            pltpu.VMEM((CH,), jnp.int32),
            pltpu.VMEM((CH, E), jnp.float32),
            pltpu.VMEM((CH, E), jnp.float32),
            pltpu.VMEM((T, E), jnp.float32),
            pltpu.SemaphoreType.DMA,
            pltpu.SemaphoreType.DMA,
        ],
    )
    def k(x_hbm, tbl_hbm, pos_hbm, out_hbm,
          idx_v0, idx_v1, rows_v0, rows_v1, pos_v, gsem0, gsem1):
        idx_b = (idx_v0, idx_v1)
        rows_b = (rows_v0, rows_v1)
        gsem_b = (gsem0, gsem1)
        wid = lax.axis_index("c") * NS + lax.axis_index("s")
        base = wid * PER_W
        bbase = wid * (PER_W // T)
        pltpu.sync_copy(pos_hbm, pos_v)

        def stage(i, b):
            off = pl.multiple_of(base + i * CH, 8)
            pltpu.sync_copy(x_hbm.at[pl.ds(off, CH)], idx_b[b])
            pltpu.async_copy(tbl_hbm.at[idx_b[b]], rows_b[b], gsem_b[b])

        def consume(i, b):
            rows_v = rows_b[b]
            pltpu.make_async_copy(
                tbl_hbm.at[idx_b[b]], rows_v, gsem_b[b]
            ).wait()

            def add_body(jrow, _):
                for jc in range(E // L):
                    pv = pos_v[jrow, pl.ds(jc * L, L)]
                    for r in range(SEQ_PER_CHUNK):
                        rr = r * T + jrow
                        rows_v[rr, pl.ds(jc * L, L)] = (
                            rows_v[rr, pl.ds(jc * L, L)] + pv
                        )
                return 0

            lax.fori_loop(0, T, add_body, 0)
            bb = bbase + i * SEQ_PER_CHUNK
            for r in range(SEQ_PER_CHUNK):
                pltpu.sync_copy(
                    rows_v.at[pl.ds(r * T, T)], out_hbm.at[bb + r]
                )

        stage(0, 0)

        def pair_body(g, _):
            i0 = 2 * g
            stage(i0 + 1, 1)
            consume(i0, 0)
            stage(i0 + 2, 0)
            consume(i0 + 1, 1)
            return 0

        lax.fori_loop(0, NCHUNK // 2 - 1, pair_body, 0)
        stage(NCHUNK - 1, 1)
        consume(NCHUNK - 2, 0)
        consume(NCHUNK - 1, 1)

    return k(xf, token_table, pos_table)


def kernel(x, token_table, pos_table):
    xf = x.reshape(N).astype(jnp.int32)
    return _sc_embed(xf, token_table, pos_table)
